# initial kernel scaffold (unmeasured)
import jax
import jax.numpy as jnp
from jax import lax
from jax.experimental import pallas as pl
from jax.experimental.pallas import tpu as pltpu

N_DEV = 4
T = 4096
D = 1024
MR = 1280


def _a2a_body(offs_ref, x_ref, cnt_ref, recv_ref, cntrecv_ref,
              dsend, drecv, csend, crecv, lsem):
    me = lax.axis_index("i")

    barrier = pltpu.get_barrier_semaphore()
    for k in (1, 2, 3):
        pl.semaphore_signal(
            barrier, inc=1,
            device_id=((me + k) % N_DEV,),
            device_id_type=pl.DeviceIdType.MESH,
        )
    pl.semaphore_wait(barrier, 3)

    o = [offs_ref[i] for i in range(N_DEV)]

    def off_of(d):
        r = o[0]
        for i in (1, 2, 3):
            r = jnp.where(d == i, o[i], r)
        return r

    lc = pltpu.make_async_copy(
        x_ref.at[pl.ds(off_of(me), MR)], recv_ref.at[0], lsem.at[0])
    lc.start()
    lcc = pltpu.make_async_copy(cnt_ref.at[0], cntrecv_ref.at[0], lsem.at[1])
    lcc.start()

    rdmas = []
    for k in (1, 2, 3):
        dst = (me + k) % N_DEV
        d = pltpu.make_async_remote_copy(
            src_ref=x_ref.at[pl.ds(off_of(dst), MR)],
            dst_ref=recv_ref.at[k],
            send_sem=dsend.at[k],
            recv_sem=drecv.at[k],
            device_id=(dst,),
            device_id_type=pl.DeviceIdType.MESH,
        )
        d.start()
        c = pltpu.make_async_remote_copy(
            src_ref=cnt_ref.at[0],
            dst_ref=cntrecv_ref.at[k],
            send_sem=csend.at[k],
            recv_sem=crecv.at[k],
            device_id=(dst,),
            device_id_type=pl.DeviceIdType.MESH,
        )
        c.start()
        rdmas.append(d)
        rdmas.append(c)

    lc.wait()
    lcc.wait()
    for r in rdmas:
        r.wait()


def kernel(x, dest):
    counts = jnp.bincount(dest, length=N_DEV).astype(jnp.int32)
    offs = (jnp.cumsum(counts) - counts).astype(jnp.int32)
    perm = jnp.argsort(dest, stable=True)
    x_sorted = x[perm].astype(jnp.bfloat16)
    x_pad = jnp.zeros((T + MR, D), jnp.bfloat16).at[:T].set(x_sorted)
    cnt_payload = jnp.zeros((1, 128), jnp.int32).at[0, :N_DEV].set(counts)

    recv, cnt_recv = pl.pallas_call(
        _a2a_body,
        out_shape=[
            jax.ShapeDtypeStruct((N_DEV, MR, D), jnp.bfloat16),
            jax.ShapeDtypeStruct((N_DEV, 128), jnp.int32),
        ],
        in_specs=[
            pl.BlockSpec(memory_space=pltpu.SMEM),
            pl.BlockSpec(memory_space=pltpu.VMEM),
            pl.BlockSpec(memory_space=pltpu.VMEM),
        ],
        out_specs=[
            pl.BlockSpec(memory_space=pltpu.VMEM),
            pl.BlockSpec(memory_space=pltpu.VMEM),
        ],
        scratch_shapes=[
            pltpu.SemaphoreType.DMA((N_DEV,)),
            pltpu.SemaphoreType.DMA((N_DEV,)),
            pltpu.SemaphoreType.DMA((N_DEV,)),
            pltpu.SemaphoreType.DMA((N_DEV,)),
            pltpu.SemaphoreType.DMA((2,)),
        ],
        compiler_params=pltpu.CompilerParams(collective_id=0),
    )(offs, x_pad, cnt_payload)

    me = lax.axis_index("i")
    slot_of_src = (me - jnp.arange(N_DEV)) % N_DEV
    c_src = jnp.take(cnt_recv, me, axis=1)[slot_of_src]
    cum = jnp.cumsum(c_src)
    starts = cum - c_src
    r = jnp.arange(T)
    p = jnp.searchsorted(cum, r, side="right")
    row_in = r - starts[p]
    idx = slot_of_src[p] * MR + row_in
    out = recv.reshape(N_DEV * MR, D)[idx]
    return out.astype(jnp.float32)


# baseline (device time: 182636 ns/iter reference)
import jax
import jax.numpy as jnp
from jax import lax
from jax.experimental import pallas as pl
from jax.experimental.pallas import tpu as pltpu

N_DEV = 4
T = 4096
D = 1024
MR = 1280


def _a2a_body(x_ref, cnt_ref, recv_ref, cntrecv_ref,
              dsend, drecv, csend, crecv, lsem):
    me = lax.axis_index("i")

    barrier = pltpu.get_barrier_semaphore()
    for k in (1, 2, 3):
        pl.semaphore_signal(
            barrier, inc=1,
            device_id=((me + k) % N_DEV,),
            device_id_type=pl.DeviceIdType.MESH,
        )
    pl.semaphore_wait(barrier, 3)

    lc = pltpu.make_async_copy(x_ref.at[me], recv_ref.at[0], lsem.at[0])
    lc.start()
    lcc = pltpu.make_async_copy(cnt_ref.at[0], cntrecv_ref.at[0], lsem.at[1])
    lcc.start()

    rdmas = []
    for k in (1, 2, 3):
        dst = (me + k) % N_DEV
        d = pltpu.make_async_remote_copy(
            src_ref=x_ref.at[dst],
            dst_ref=recv_ref.at[k],
            send_sem=dsend.at[k],
            recv_sem=drecv.at[k],
            device_id=(dst,),
            device_id_type=pl.DeviceIdType.MESH,
        )
        d.start()
        c = pltpu.make_async_remote_copy(
            src_ref=cnt_ref.at[0],
            dst_ref=cntrecv_ref.at[k],
            send_sem=csend.at[k],
            recv_sem=crecv.at[k],
            device_id=(dst,),
            device_id_type=pl.DeviceIdType.MESH,
        )
        c.start()
        rdmas.append(d)
        rdmas.append(c)

    lc.wait()
    lcc.wait()
    for r in rdmas:
        r.wait()


def kernel(x, dest):
    counts = jnp.bincount(dest, length=N_DEV).astype(jnp.int32)
    offs = jnp.cumsum(counts) - counts
    perm = jnp.argsort(dest, stable=True)
    src_idx = jnp.clip(offs[:, None] + jnp.arange(MR)[None, :], 0, T - 1)
    x_chunks = x[perm[src_idx]].astype(jnp.bfloat16)
    cnt_payload = jnp.zeros((1, 128), jnp.int32).at[0, :N_DEV].set(counts)

    recv, cnt_recv = pl.pallas_call(
        _a2a_body,
        out_shape=[
            jax.ShapeDtypeStruct((N_DEV, MR, D), jnp.bfloat16),
            jax.ShapeDtypeStruct((N_DEV, 128), jnp.int32),
        ],
        in_specs=[
            pl.BlockSpec(memory_space=pltpu.VMEM),
            pl.BlockSpec(memory_space=pltpu.VMEM),
        ],
        out_specs=[
            pl.BlockSpec(memory_space=pltpu.VMEM),
            pl.BlockSpec(memory_space=pltpu.VMEM),
        ],
        scratch_shapes=[
            pltpu.SemaphoreType.DMA((N_DEV,)),
            pltpu.SemaphoreType.DMA((N_DEV,)),
            pltpu.SemaphoreType.DMA((N_DEV,)),
            pltpu.SemaphoreType.DMA((N_DEV,)),
            pltpu.SemaphoreType.DMA((2,)),
        ],
        compiler_params=pltpu.CompilerParams(collective_id=0),
    )(x_chunks, cnt_payload)

    me = lax.axis_index("i")
    slot_of_src = (me - jnp.arange(N_DEV)) % N_DEV
    c_src = jnp.take(cnt_recv, me, axis=1)[slot_of_src]
    cum = jnp.cumsum(c_src)
    starts = cum - c_src
    r = jnp.arange(T)
    p = jnp.searchsorted(cum, r, side="right")
    row_in = r - starts[p]
    idx = slot_of_src[p] * MR + row_in
    out = recv.reshape(N_DEV * MR, D)[idx]
    return out.astype(jnp.float32)
